# baseline (speedup 1.0000x reference)
import jax
import jax.numpy as jnp
from jax.experimental import pallas as pl
from jax.experimental.pallas import tpu as pltpu

D = 512
K = 64


def _netvlad_kernel(x_ref, w_ref, b_ref, c_ref, out_ref):
    x = x_ref[0]
    w = w_ref[...]
    b = b_ref[...]
    c = c_ref[...]

    logits = jnp.dot(w, x, preferred_element_type=jnp.float32) + b

    m = jnp.max(logits, axis=0, keepdims=True)
    e = jnp.exp(logits - m)
    alpha = e / jnp.sum(e, axis=0, keepdims=True)

    s = jnp.sum(alpha, axis=1, keepdims=True)
    vlad = jax.lax.dot_general(
        x, alpha, (((1,), (1,)), ((), ())),
        preferred_element_type=jnp.float32)
    vlad = vlad - c * s.reshape(1, K)

    ssq = jnp.sum(vlad * vlad, axis=0, keepdims=True)
    vlad = vlad * jax.lax.rsqrt(ssq)
    gsq = jnp.sum(vlad * vlad, axis=(0, 1), keepdims=True)
    out_ref[0] = vlad * jax.lax.rsqrt(gsq)


def kernel(inputs, conv_w, conv_b, centers):
    B, d, H, W = inputs.shape
    N = H * W
    x = inputs.reshape(B, d, N)
    out = pl.pallas_call(
        _netvlad_kernel,
        grid=(B,),
        in_specs=[
            pl.BlockSpec((1, d, N), lambda b: (b, 0, 0)),
            pl.BlockSpec((K, d), lambda b: (0, 0)),
            pl.BlockSpec((K, 1), lambda b: (0, 0)),
            pl.BlockSpec((d, K), lambda b: (0, 0)),
        ],
        out_specs=pl.BlockSpec((1, d, K), lambda b: (b, 0, 0)),
        out_shape=jax.ShapeDtypeStruct((B, d, K), jnp.float32),
        compiler_params=pltpu.CompilerParams(
            dimension_semantics=("parallel",),
            vmem_limit_bytes=48 * 1024 * 1024,
        ),
    )(x, conv_w, conv_b.reshape(K, 1), centers)
    return out.reshape(B, d * K)
